# baseline (speedup 1.0000x reference)
import functools

import jax
import jax.numpy as jnp
from jax import lax
from jax.experimental import pallas as pl
from jax.experimental.pallas import tpu as pltpu
from jax.experimental.pallas import tpu_sc as plsc

N = 819200
NOFF = 16384
TOTAL = 2 * N
NC = 2
NS = 16
NW = NC * NS
CHUNK = N // NW
OCHUNK = NOFF // NW
LANES = 16

_OUT_TYPE = (
    jax.ShapeDtypeStruct((TOTAL,), jnp.int32),
    jax.ShapeDtypeStruct((2 * NOFF + 1,), jnp.int32),
    jax.ShapeDtypeStruct((TOTAL,), jnp.float32),
)

_SCRATCH = [
    pltpu.VMEM((CHUNK,), jnp.int32),
    pltpu.VMEM((CHUNK,), jnp.int32),
    pltpu.VMEM((CHUNK,), jnp.float32),
    pltpu.VMEM((CHUNK,), jnp.float32),
    pltpu.VMEM((OCHUNK,), jnp.int32),
    pltpu.VMEM((OCHUNK + LANES,), jnp.int32),
]

_MESH = plsc.VectorSubcoreMesh(core_axis_name="c", subcore_axis_name="s")


@functools.partial(
    pl.kernel,
    out_type=_OUT_TYPE,
    mesh=_MESH,
    scratch_types=_SCRATCH,
)
def _tbe_prepare(ind0, ind1, off0, off1, psw0, psw1,
                 out_ind, out_off, out_psw,
                 bi0, bi1, bf0, bf1, bo0, bo1):
    wid = lax.axis_index("s") * NC + lax.axis_index("c")
    base = wid * CHUNK

    pltpu.sync_copy(ind0.at[pl.ds(base, CHUNK)], bi0)
    pltpu.sync_copy(bi0, out_ind.at[pl.ds(base, CHUNK)])
    pltpu.sync_copy(ind1.at[pl.ds(base, CHUNK)], bi1)
    pltpu.sync_copy(bi1, out_ind.at[pl.ds(N + base, CHUNK)])

    pltpu.sync_copy(psw0.at[pl.ds(base, CHUNK)], bf0)
    pltpu.sync_copy(bf0, out_psw.at[pl.ds(base, CHUNK)])
    pltpu.sync_copy(psw1.at[pl.ds(base, CHUNK)], bf1)
    pltpu.sync_copy(bf1, out_psw.at[pl.ds(N + base, CHUNK)])

    obase = wid * OCHUNK
    pltpu.sync_copy(off0.at[pl.ds(obase, OCHUNK)], bo0)
    pltpu.sync_copy(bo0, out_off.at[pl.ds(obase, OCHUNK)])

    pltpu.sync_copy(off1.at[pl.ds(obase, OCHUNK)], bo1.at[pl.ds(0, OCHUNK)])
    for i in range(OCHUNK // LANES):
        sl = pl.ds(i * LANES, LANES)
        bo1[sl] = bo1[sl] + jnp.int32(N)
    bo1[pl.ds(OCHUNK, LANES)] = jnp.full((LANES,), TOTAL, dtype=jnp.int32)

    @pl.when(wid == NW - 1)
    def _():
        pltpu.sync_copy(bo1.at[pl.ds(0, OCHUNK + 1)],
                        out_off.at[pl.ds(NOFF + obase, OCHUNK + 1)])

    @pl.when(wid != NW - 1)
    def _():
        pltpu.sync_copy(bo1.at[pl.ds(0, OCHUNK)],
                        out_off.at[pl.ds(NOFF + obase, OCHUNK)])


def kernel(indices_0, indices_1, offsets_0, offsets_1,
           per_sample_weights_0, per_sample_weights_1):
    return _tbe_prepare(indices_0.astype(jnp.int32),
                        indices_1.astype(jnp.int32),
                        offsets_0, offsets_1,
                        per_sample_weights_0, per_sample_weights_1)
